# baseline (speedup 1.0000x reference)
import functools

import jax
import jax.numpy as jnp
from jax import lax
from jax.experimental import pallas as pl
from jax.experimental.pallas import tpu as pltpu
from jax.experimental.pallas import tpu_sc as plsc

B = 16
T = 2048
DIM = 64
H = 8
NBUCK = 32
NP = B * H
CB = 4
QB = CB * 64
KB = QB + 64
SELF_VAL = -50000.0


@functools.cache
def _mesh():
    return plsc.VectorSubcoreMesh(core_axis_name="c", subcore_axis_name="s")


def _hash_body(qk_ref, v_ref, rot_ref, out_ref, qkv_ref):
    rot = rot_ref[0].reshape(DIM, 128)
    cols = []
    for bb in range(2):
        q = qk_ref[bb]
        qkv_ref[bb] = jnp.concatenate([q, v_ref[bb]], axis=1)
        r = jnp.dot(q, rot, preferred_element_type=jnp.float32)
        for h in range(H):
            w = r[:, h * 16:(h + 1) * 16]
            m = jnp.max(jnp.abs(w), axis=1, keepdims=True)
            io = lax.broadcasted_iota(jnp.int32, (T, 16), 1)
            cand = jnp.where(w == m, io,
                             jnp.where(w == -m, io + 16, 2 * NBUCK))
            cols.append(jnp.min(cand, axis=1, keepdims=True))
    out_ref[0] = jnp.concatenate(cols, axis=1)


def _hash_call(qk, v, rotations):
    return pl.pallas_call(
        _hash_body,
        grid=(8,),
        in_specs=[
            pl.BlockSpec((2, T, DIM), lambda g: (g, 0, 0)),
            pl.BlockSpec((2, T, DIM), lambda g: (g, 0, 0)),
            pl.BlockSpec((1, DIM, H, 16), lambda g: (0, 0, 0, 0)),
        ],
        out_specs=[
            pl.BlockSpec((1, T, 16), lambda g: (g, 0, 0)),
            pl.BlockSpec((2, T, 128), lambda g: (g, 0, 0)),
        ],
        out_shape=[
            jax.ShapeDtypeStruct((8, T, 16), jnp.int32),
            jax.ShapeDtypeStruct((B, T, 128), jnp.float32),
        ],
    )(qk, v, rotations)


_GC2 = 128


def _sort_body(bT_hbm, qkv_hbm, st_hbm, pos_hbm, sqv_hbm,
               bkT, stpm, pospm, hist, idxv, rows, sem):
    cid = lax.axis_index("c")
    sid = lax.axis_index("s")

    @pl.when(sid < 4)
    def _():
        g = cid * 4 + sid
        lane = jnp.arange(16, dtype=jnp.int32)
        loff = lane * NBUCK
        boff = (2 * g + lane // 8) * T
        poff = (g * 16 + lane) * T
        pltpu.sync_copy(bT_hbm.at[g], bkT)

        def zero(i, carry):
            hist[pl.ds(i * 16, 16)] = jnp.zeros((16,), jnp.int32)
            return carry
        lax.fori_loop(0, NBUCK * 16 // 16, zero, 0)

        def pass1(t, carry):
            bk = bkT[pl.ds(t * 16, 16)]
            idx = loff + bk
            c = plsc.load_gather(hist, [idx])
            bkT[pl.ds(t * 16, 16)] = bk | (c << 5)
            plsc.store_scatter(hist, [idx], c + 1)
            return carry
        lax.fori_loop(0, T, pass1, 0)

        def scan(bb, run):
            ii = loff + bb
            cnt = plsc.load_gather(hist, [ii])
            plsc.store_scatter(hist, [ii], run)
            return run + cnt
        lax.fori_loop(0, NBUCK, scan, jnp.zeros((16,), jnp.int32))

        def pass2(t, carry):
            w = bkT[pl.ds(t * 16, 16)]
            bk = w & (NBUCK - 1)
            r = w >> 5
            base = plsc.load_gather(hist, [loff + bk])
            posv = base + r
            tv = jnp.full((16,), t, jnp.int32)
            plsc.store_scatter(pospm, [lane, tv], posv + poff)
            plsc.store_scatter(stpm, [lane, posv], tv + boff)
            return carry
        lax.fori_loop(0, T, pass2, 0)

        pltpu.sync_copy(stpm, st_hbm.at[pl.ds(g * 16, 16), :])
        pltpu.sync_copy(pospm, pos_hbm.at[pl.ds(g * 16, 16), :])

    plsc.subcore_barrier()

    for pi in range(4):
        p = cid * 64 + sid * 4 + pi
        for c in range(T // _GC2):
            pltpu.sync_copy(st_hbm.at[p, pl.ds(c * _GC2, _GC2)], idxv)
            pltpu.async_copy(qkv_hbm.at[idxv], rows, sem).wait()
            pltpu.sync_copy(rows, sqv_hbm.at[p, pl.ds(c * _GC2, _GC2), :])


def _sort_call(bT, qkv_flat):
    return pl.kernel(
        _sort_body,
        out_type=[
            jax.ShapeDtypeStruct((NP, T), jnp.int32),
            jax.ShapeDtypeStruct((NP, T), jnp.int32),
            jax.ShapeDtypeStruct((NP, T, 128), jnp.float32),
        ],
        mesh=_mesh(),
        compiler_params=pltpu.CompilerParams(needs_layout_passes=False),
        scratch_types=[
            pltpu.VMEM((T * 16,), jnp.int32),
            pltpu.VMEM((16, T), jnp.int32),
            pltpu.VMEM((16, T), jnp.int32),
            pltpu.VMEM((NBUCK * 16,), jnp.int32),
            pltpu.VMEM((_GC2,), jnp.int32),
            pltpu.VMEM((_GC2, 128), jnp.float32),
            pltpu.SemaphoreType.DMA,
        ],
    )(bT, qkv_flat)


_GCH = 512


def _rowgather_body(tab_hbm, idx_hbm, out_hbm, idxv, rows, sem):
    wid = lax.axis_index("s") * 2 + lax.axis_index("c")
    for pi in range(4):
        p = wid * 4 + pi
        for c in range(T // _GCH):
            pltpu.sync_copy(idx_hbm.at[p, pl.ds(c * _GCH, _GCH)], idxv)
            pltpu.async_copy(tab_hbm.at[idxv], rows, sem).wait()
            pltpu.sync_copy(rows, out_hbm.at[p, pl.ds(c * _GCH, _GCH), :])


def _rowgather_call(tab, idx):
    return pl.kernel(
        _rowgather_body,
        out_type=jax.ShapeDtypeStruct((NP, T, 128), jnp.float32),
        mesh=_mesh(),
        scratch_types=[
            pltpu.VMEM((_GCH,), jnp.int32),
            pltpu.VMEM((_GCH, 128), jnp.float32),
            pltpu.SemaphoreType.DMA,
        ],
    )(tab, idx)


NG = 8
SB = NG * QB


def _attn_group(q, kraw, vall, tq, tk):
    f32 = jnp.float32
    nrm2 = lax.dot_general(kraw * kraw, jnp.ones((DIM, 1), f32),
                           (((1,), (0,)), ((), ())),
                           preferred_element_type=f32)
    inv = (DIM ** -0.5) / jnp.maximum(jnp.sqrt(nrm2), 1e-12)
    k = kraw * inv
    dots = lax.dot_general(q, k, (((1,), (1,)), ((), ())),
                           preferred_element_type=f32)
    qj = lax.broadcasted_iota(jnp.int32, (QB, KB), 0) // 64
    col = lax.broadcasted_iota(jnp.int32, (QB, KB), 1)
    inband = (col >= 64 * qj) & (col < 64 * qj + 128)
    neg = -jnp.finfo(jnp.float32).max
    dots = jnp.where(tq < tk, neg, dots)
    dots = jnp.where(tq == tk, SELF_VAL, dots)
    dots = jnp.where(inband, dots, neg)
    m = jnp.max(dots, axis=1, keepdims=True)
    e = jnp.exp(dots - m)
    s = jnp.sum(e, axis=1, keepdims=True)
    o = lax.dot_general(e, vall, (((1,), (0,)), ((), ())),
                        preferred_element_type=f32) / s
    lse = m + jnp.log(s)
    pad = jnp.zeros((QB, 128 - DIM - 1), f32)
    return jnp.concatenate([o, lse, pad], axis=1)


def _attn_body(qv_ref, qvp_ref, stq_ref, stkw_ref, stkp_ref, so_ref):
    qv = qv_ref[0]
    tq = stq_ref[0, 0]
    tkw = stkw_ref[0, 0]
    outs = []
    for g in range(NG):
        r0 = g * QB
        q = qv[r0:r0 + QB, :DIM]
        tqg = tq[r0:r0 + QB]
        if g == 0:
            kraw = jnp.concatenate([qvp_ref[0, :, :DIM], q], axis=0)
            vall = jnp.concatenate([qvp_ref[0, :, DIM:], qv[r0:r0 + QB, DIM:]],
                                   axis=0)
            tkg = jnp.concatenate([stkp_ref[0, 0], tkw[:, :QB]], axis=1)
        else:
            kraw = qv[r0 - 64:r0 + QB, :DIM]
            vall = qv[r0 - 64:r0 + QB, DIM:]
            tkg = tkw[:, r0 - 64:r0 + QB]
        outs.append(_attn_group(q, kraw, vall, tqg, tkg))
    so_ref[0] = jnp.concatenate(outs, axis=0)


def _attn_call(sqv, stq, stkw, stkp):
    nI = T * H // SB
    prev = lambda b, i: (b, (NG * CB * i + 255) % 256, 0)
    prev4 = lambda b, i: (b, (NG * CB * i + 255) % 256, 0, 0)
    return pl.pallas_call(
        _attn_body,
        grid=(B, nI),
        in_specs=[
            pl.BlockSpec((1, SB, 128), lambda b, i: (b, i, 0)),
            pl.BlockSpec((1, 64, 128), prev),
            pl.BlockSpec((1, 1, SB, 1), lambda b, i: (b, i, 0, 0)),
            pl.BlockSpec((1, 1, 1, SB), lambda b, i: (b, i, 0, 0)),
            pl.BlockSpec((1, 1, 1, 64), prev4),
        ],
        out_specs=pl.BlockSpec((1, SB, 128), lambda b, i: (b, i, 0)),
        out_shape=jax.ShapeDtypeStruct((B, T * H, 128), jnp.float32),
    )(sqv, sqv, stq, stkw, stkp)


_TCH = 64


def _unsort_combine_body(so_hbm, pos_hbm, out_hbm, idx, rows, pbuf, outb, sem):
    wid = lax.axis_index("s") * 2 + lax.axis_index("c")
    b = wid // 2
    t0 = (wid % 2) * (T // 2)
    lane = jnp.arange(16, dtype=jnp.int32)

    def chunk(c, carry):
        t0c = t0 + c * _TCH
        pltpu.sync_copy(
            pos_hbm.at[pl.ds(b * 8, 8), pl.ds(t0 + (c // 2) * 128, 128)],
            idx)
        sub = (c % 2) * _TCH
        cps = [pltpu.async_copy(so_hbm.at[idx.at[h, pl.ds(sub, _TCH)]],
                                rows.at[h], sem)
               for h in range(H)]
        for cp in cps:
            cp.wait()

        def wgrp(g, carry2):
            tv = g * 16 + lane
            lg = [plsc.load_gather(
                rows, [jnp.full((16,), h, jnp.int32), tv,
                       jnp.full((16,), DIM, jnp.int32)]) for h in range(H)]
            m = lg[0]
            for h in range(1, H):
                m = jnp.maximum(m, lg[h])
            e = [jnp.exp(x - m) for x in lg]
            s = e[0]
            for h in range(1, H):
                s = s + e[h]
            for h in range(H):
                pbuf[h, pl.ds(g * 16, 16)] = e[h] / s
            return carry2
        lax.fori_loop(0, _TCH // 16, wgrp, 0)

        def accum(t, carry2):
            accs = [jnp.zeros((16,), jnp.float32) for _ in range(DIM // 16)]
            for h in range(H):
                pv = plsc.load_gather(
                    pbuf, [jnp.full((16,), h, jnp.int32),
                           jnp.full((16,), t, jnp.int32)])
                for j in range(DIM // 16):
                    accs[j] = accs[j] + pv * rows[h, t, pl.ds(j * 16, 16)]
            for j in range(DIM // 16):
                outb[t, pl.ds(j * 16, 16)] = accs[j]
            return carry2
        lax.fori_loop(0, _TCH, accum, 0)
        pltpu.sync_copy(outb, out_hbm.at[b, pl.ds(t0c, _TCH), :])
        return carry

    lax.fori_loop(0, (T // 2) // _TCH, chunk, 0)


def _unsort_combine_call(so_flat, pos):
    return pl.kernel(
        _unsort_combine_body,
        out_type=jax.ShapeDtypeStruct((B, T, DIM), jnp.float32),
        mesh=_mesh(),
        compiler_params=pltpu.CompilerParams(needs_layout_passes=False),
        scratch_types=[
            pltpu.VMEM((H, 128), jnp.int32),
            pltpu.VMEM((H, _TCH, 128), jnp.float32),
            pltpu.VMEM((H, _TCH), jnp.float32),
            pltpu.VMEM((_TCH, DIM), jnp.float32),
            pltpu.SemaphoreType.DMA,
        ],
    )(so_flat, pos)


def _combine_body(o_ref, out_ref):
    o = o_ref[0]
    lg = o[:, :, DIM:DIM + 1]
    m = jnp.max(lg, axis=0, keepdims=True)
    e = jnp.exp(lg - m)
    s = jnp.sum(e, axis=0, keepdims=True)
    p = e / s
    out_ref[0] = jnp.sum(o[:, :, :DIM] * p, axis=0)


def _combine_call(o):
    TC = 256
    nb = o.shape[0]
    return pl.pallas_call(
        _combine_body,
        grid=(nb, T // TC),
        in_specs=[
            pl.BlockSpec((1, H, TC, 128), lambda b, c: (b, 0, c, 0)),
        ],
        out_specs=pl.BlockSpec((1, TC, DIM), lambda b, c: (b, c, 0)),
        out_shape=jax.ShapeDtypeStruct((nb, T, DIM), jnp.float32),
    )(o)


def kernel(qk, v, rotations):
    bT, qkv = _hash_call(qk, v, rotations)
    st, pos, sqv = _sort_call(bT.reshape(8, T * 16), qkv.reshape(B * T, 128))
    st_flat = st.reshape(B, T * H)
    stq = st_flat.reshape(B, T * H // SB, SB, 1)
    stkw = st_flat.reshape(B, T * H // SB, 1, SB)
    stkp = st_flat.reshape(B, 256, 1, 64)
    so = _attn_call(sqv.reshape(B, T * H, 128), stq, stkw, stkp)
    return _unsort_combine_call(so.reshape(NP * T, 128), pos)


# speedup vs baseline: 1.0768x; 1.0768x over previous
import functools

import jax
import jax.numpy as jnp
from jax import lax
from jax.experimental import pallas as pl
from jax.experimental.pallas import tpu as pltpu
from jax.experimental.pallas import tpu_sc as plsc

B = 16
T = 2048
DIM = 64
H = 8
NBUCK = 32
NP = B * H
CB = 4
QB = CB * 64
KB = QB + 64
SELF_VAL = -50000.0


@functools.cache
def _mesh():
    return plsc.VectorSubcoreMesh(core_axis_name="c", subcore_axis_name="s")


def _hash_body(qk_ref, v_ref, rot_ref, out_ref, qkv_ref):
    rot = rot_ref[0].reshape(DIM, 128)
    cols = []
    for bb in range(2):
        q = qk_ref[bb]
        qkv_ref[bb] = jnp.concatenate([q, v_ref[bb]], axis=1)
        r = jnp.dot(q, rot, preferred_element_type=jnp.float32)
        for h in range(H):
            w = r[:, h * 16:(h + 1) * 16]
            m = jnp.max(jnp.abs(w), axis=1, keepdims=True)
            io = lax.broadcasted_iota(jnp.int32, (T, 16), 1)
            cand = jnp.where(w == m, io,
                             jnp.where(w == -m, io + 16, 2 * NBUCK))
            cols.append(jnp.min(cand, axis=1, keepdims=True))
    out_ref[0] = jnp.concatenate(cols, axis=1)


def _hash_call(qk, v, rotations):
    return pl.pallas_call(
        _hash_body,
        grid=(8,),
        in_specs=[
            pl.BlockSpec((2, T, DIM), lambda g: (g, 0, 0)),
            pl.BlockSpec((2, T, DIM), lambda g: (g, 0, 0)),
            pl.BlockSpec((1, DIM, H, 16), lambda g: (0, 0, 0, 0)),
        ],
        out_specs=[
            pl.BlockSpec((1, T, 16), lambda g: (g, 0, 0)),
            pl.BlockSpec((2, T, 128), lambda g: (g, 0, 0)),
        ],
        out_shape=[
            jax.ShapeDtypeStruct((8, T, 16), jnp.int32),
            jax.ShapeDtypeStruct((B, T, 128), jnp.float32),
        ],
    )(qk, v, rotations)


def _sort_body(bT_hbm, st_hbm, pos_hbm, bkT, stpm, pospm, hist):
    wid = lax.axis_index("s") * 2 + lax.axis_index("c")

    @pl.when(wid < 8)
    def _():
        g = wid
        lane = jnp.arange(16, dtype=jnp.int32)
        loff = lane * NBUCK
        boff = (2 * g + lane // 8) * T
        poff = (g * 16 + lane) * T
        pltpu.sync_copy(bT_hbm.at[g], bkT)

        def zero(i, carry):
            hist[pl.ds(i * 16, 16)] = jnp.zeros((16,), jnp.int32)
            return carry
        lax.fori_loop(0, NBUCK * 16 // 16, zero, 0)

        def pass1(t, carry):
            bk = bkT[pl.ds(t * 16, 16)]
            idx = loff + bk
            c = plsc.load_gather(hist, [idx])
            bkT[pl.ds(t * 16, 16)] = bk | (c << 5)
            plsc.store_scatter(hist, [idx], c + 1)
            return carry
        lax.fori_loop(0, T, pass1, 0)

        def scan(bb, run):
            ii = loff + bb
            cnt = plsc.load_gather(hist, [ii])
            plsc.store_scatter(hist, [ii], run)
            return run + cnt
        lax.fori_loop(0, NBUCK, scan, jnp.zeros((16,), jnp.int32))

        def pass2(t, carry):
            w = bkT[pl.ds(t * 16, 16)]
            bk = w & (NBUCK - 1)
            r = w >> 5
            base = plsc.load_gather(hist, [loff + bk])
            posv = base + r
            tv = jnp.full((16,), t, jnp.int32)
            plsc.store_scatter(pospm, [lane, tv], posv + poff)
            plsc.store_scatter(stpm, [lane, posv], tv + boff)
            return carry
        lax.fori_loop(0, T, pass2, 0)

        pltpu.sync_copy(stpm, st_hbm.at[pl.ds(g * 16, 16), :])
        pltpu.sync_copy(pospm, pos_hbm.at[pl.ds(g * 16, 16), :])


def _sort_call(bT):
    return pl.kernel(
        _sort_body,
        out_type=[
            jax.ShapeDtypeStruct((NP, T), jnp.int32),
            jax.ShapeDtypeStruct((NP, T), jnp.int32),
        ],
        mesh=_mesh(),
        compiler_params=pltpu.CompilerParams(needs_layout_passes=False),
        scratch_types=[
            pltpu.VMEM((T * 16,), jnp.int32),
            pltpu.VMEM((16, T), jnp.int32),
            pltpu.VMEM((16, T), jnp.int32),
            pltpu.VMEM((NBUCK * 16,), jnp.int32),
        ],
    )(bT)


_GCH = 256


def _rowgather_body(tab_hbm, idx_hbm, out_hbm, idxv0, idxv1, rows0, rows1,
                    sem0, sem1):
    wid = lax.axis_index("s") * 2 + lax.axis_index("c")
    bufs = [(idxv0, rows0, sem0), (idxv1, rows1, sem1)]
    nch = T // _GCH
    for pi in range(4):
        p = wid * 4 + pi
        cps = [None, None]
        for c in range(nch):
            ib, rb, sm = bufs[c % 2]
            pltpu.sync_copy(idx_hbm.at[p, pl.ds(c * _GCH, _GCH)], ib)
            cps[c % 2] = pltpu.async_copy(tab_hbm.at[ib], rb, sm)
            if c >= 1:
                cps[(c - 1) % 2].wait()
                prb = bufs[(c - 1) % 2][1]
                pltpu.sync_copy(
                    prb, out_hbm.at[p, pl.ds((c - 1) * _GCH, _GCH), :])
        cps[(nch - 1) % 2].wait()
        pltpu.sync_copy(bufs[(nch - 1) % 2][1],
                        out_hbm.at[p, pl.ds((nch - 1) * _GCH, _GCH), :])


def _rowgather_call(tab, idx):
    return pl.kernel(
        _rowgather_body,
        out_type=jax.ShapeDtypeStruct((NP, T, 128), jnp.float32),
        mesh=_mesh(),
        scratch_types=[
            pltpu.VMEM((_GCH,), jnp.int32),
            pltpu.VMEM((_GCH,), jnp.int32),
            pltpu.VMEM((_GCH, 128), jnp.float32),
            pltpu.VMEM((_GCH, 128), jnp.float32),
            pltpu.SemaphoreType.DMA,
            pltpu.SemaphoreType.DMA,
        ],
    )(tab, idx)


NG = 8
SB = NG * QB


def _attn_group(q, kraw, vall, tq, tk):
    f32 = jnp.float32
    nrm2 = lax.dot_general(kraw * kraw, jnp.ones((DIM, 1), f32),
                           (((1,), (0,)), ((), ())),
                           preferred_element_type=f32)
    inv = (DIM ** -0.5) / jnp.maximum(jnp.sqrt(nrm2), 1e-12)
    k = kraw * inv
    dots = lax.dot_general(q, k, (((1,), (1,)), ((), ())),
                           preferred_element_type=f32)
    qj = lax.broadcasted_iota(jnp.int32, (QB, KB), 0) // 64
    col = lax.broadcasted_iota(jnp.int32, (QB, KB), 1)
    inband = (col >= 64 * qj) & (col < 64 * qj + 128)
    neg = -jnp.finfo(jnp.float32).max
    dots = jnp.where(tq < tk, neg, dots)
    dots = jnp.where(tq == tk, SELF_VAL, dots)
    dots = jnp.where(inband, dots, neg)
    m = jnp.max(dots, axis=1, keepdims=True)
    e = jnp.exp(dots - m)
    s = jnp.sum(e, axis=1, keepdims=True)
    o = lax.dot_general(e, vall, (((1,), (0,)), ((), ())),
                        preferred_element_type=f32) / s
    lse = m + jnp.log(s)
    pad = jnp.zeros((QB, 128 - DIM - 1), f32)
    return jnp.concatenate([o, lse, pad], axis=1)


def _attn_body(qv_ref, qvp_ref, stq_ref, stkw_ref, stkp_ref, so_ref):
    qv = qv_ref[0]
    tq = stq_ref[0, 0]
    tkw = stkw_ref[0, 0]
    outs = []
    for g in range(NG):
        r0 = g * QB
        q = qv[r0:r0 + QB, :DIM]
        tqg = tq[r0:r0 + QB]
        if g == 0:
            kraw = jnp.concatenate([qvp_ref[0, :, :DIM], q], axis=0)
            vall = jnp.concatenate([qvp_ref[0, :, DIM:], qv[r0:r0 + QB, DIM:]],
                                   axis=0)
            tkg = jnp.concatenate([stkp_ref[0, 0], tkw[:, :QB]], axis=1)
        else:
            kraw = qv[r0 - 64:r0 + QB, :DIM]
            vall = qv[r0 - 64:r0 + QB, DIM:]
            tkg = tkw[:, r0 - 64:r0 + QB]
        outs.append(_attn_group(q, kraw, vall, tqg, tkg))
    so_ref[0] = jnp.concatenate(outs, axis=0)


def _attn_call(sqv, stq, stkw, stkp):
    nI = T * H // SB
    prev = lambda b, i: (b, (NG * CB * i + 255) % 256, 0)
    prev4 = lambda b, i: (b, (NG * CB * i + 255) % 256, 0, 0)
    return pl.pallas_call(
        _attn_body,
        grid=(B, nI),
        in_specs=[
            pl.BlockSpec((1, SB, 128), lambda b, i: (b, i, 0)),
            pl.BlockSpec((1, 64, 128), prev),
            pl.BlockSpec((1, 1, SB, 1), lambda b, i: (b, i, 0, 0)),
            pl.BlockSpec((1, 1, 1, SB), lambda b, i: (b, i, 0, 0)),
            pl.BlockSpec((1, 1, 1, 64), prev4),
        ],
        out_specs=pl.BlockSpec((1, SB, 128), lambda b, i: (b, i, 0)),
        out_shape=jax.ShapeDtypeStruct((B, T * H, 128), jnp.float32),
    )(sqv, sqv, stq, stkw, stkp)


_TCH = 64


def _unsort_combine_body(so_hbm, pos_hbm, out_hbm, idx, rows, pbuf, outb, sem):
    wid = lax.axis_index("s") * 2 + lax.axis_index("c")
    b = wid // 2
    t0 = (wid % 2) * (T // 2)
    lane = jnp.arange(16, dtype=jnp.int32)

    def chunk(c, carry):
        t0c = t0 + c * _TCH
        pltpu.sync_copy(
            pos_hbm.at[pl.ds(b * 8, 8), pl.ds(t0 + (c // 2) * 128, 128)],
            idx)
        sub = (c % 2) * _TCH
        cps = [pltpu.async_copy(so_hbm.at[idx.at[h, pl.ds(sub, _TCH)]],
                                rows.at[h], sem)
               for h in range(H)]
        for cp in cps:
            cp.wait()

        def wgrp(g, carry2):
            tv = g * 16 + lane
            lg = [plsc.load_gather(
                rows, [jnp.full((16,), h, jnp.int32), tv,
                       jnp.full((16,), DIM, jnp.int32)]) for h in range(H)]
            m = lg[0]
            for h in range(1, H):
                m = jnp.maximum(m, lg[h])
            e = [jnp.exp(x - m) for x in lg]
            s = e[0]
            for h in range(1, H):
                s = s + e[h]
            for h in range(H):
                pbuf[h, pl.ds(g * 16, 16)] = e[h] / s
            return carry2
        lax.fori_loop(0, _TCH // 16, wgrp, 0)

        def accum(t, carry2):
            accs = [jnp.zeros((16,), jnp.float32) for _ in range(DIM // 16)]
            for h in range(H):
                pv = plsc.load_gather(
                    pbuf, [jnp.full((16,), h, jnp.int32),
                           jnp.full((16,), t, jnp.int32)])
                for j in range(DIM // 16):
                    accs[j] = accs[j] + pv * rows[h, t, pl.ds(j * 16, 16)]
            for j in range(DIM // 16):
                outb[t, pl.ds(j * 16, 16)] = accs[j]
            return carry2
        lax.fori_loop(0, _TCH, accum, 0)
        pltpu.sync_copy(outb, out_hbm.at[b, pl.ds(t0c, _TCH), :])
        return carry

    lax.fori_loop(0, (T // 2) // _TCH, chunk, 0)


def _unsort_combine_call(so_flat, pos):
    return pl.kernel(
        _unsort_combine_body,
        out_type=jax.ShapeDtypeStruct((B, T, DIM), jnp.float32),
        mesh=_mesh(),
        compiler_params=pltpu.CompilerParams(needs_layout_passes=False),
        scratch_types=[
            pltpu.VMEM((H, 128), jnp.int32),
            pltpu.VMEM((H, _TCH, 128), jnp.float32),
            pltpu.VMEM((H, _TCH), jnp.float32),
            pltpu.VMEM((_TCH, DIM), jnp.float32),
            pltpu.SemaphoreType.DMA,
        ],
    )(so_flat, pos)


def _combine_body(o_ref, out_ref):
    o = o_ref[0]
    lg = o[:, :, DIM:DIM + 1]
    m = jnp.max(lg, axis=0, keepdims=True)
    e = jnp.exp(lg - m)
    s = jnp.sum(e, axis=0, keepdims=True)
    p = e / s
    out_ref[0] = jnp.sum(o[:, :, :DIM] * p, axis=0)


def _combine_call(o):
    TC = 256
    nb = o.shape[0]
    return pl.pallas_call(
        _combine_body,
        grid=(nb, T // TC),
        in_specs=[
            pl.BlockSpec((1, H, TC, 128), lambda b, c: (b, 0, c, 0)),
        ],
        out_specs=pl.BlockSpec((1, TC, DIM), lambda b, c: (b, c, 0)),
        out_shape=jax.ShapeDtypeStruct((nb, T, DIM), jnp.float32),
    )(o)


def kernel(qk, v, rotations):
    bT, qkv = _hash_call(qk, v, rotations)
    st, pos = _sort_call(bT.reshape(8, T * 16))
    sqv = _rowgather_call(qkv.reshape(B * T, 128), st)
    st_flat = st.reshape(B, T * H)
    stq = st_flat.reshape(B, T * H // SB, SB, 1)
    stkw = st_flat.reshape(B, T * H // SB, 1, SB)
    stkp = st_flat.reshape(B, 256, 1, 64)
    so = _attn_call(sqv.reshape(B, T * H, 128), stq, stkw, stkp)
    return _unsort_combine_call(so.reshape(NP * T, 128), pos)
